# baseline (speedup 1.0000x reference)
import functools

import jax
import jax.numpy as jnp
from jax import lax
from jax.experimental import pallas as pl
from jax.experimental.pallas import tpu as pltpu
from jax.experimental.pallas import tpu_sc as plsc

_NC = 2
_NS = 16
_NW = _NC * _NS
_C = 80


def _linear_body(h_ref, w_ref, b_ref, o_ref):
    o_ref[...] = (
        jnp.dot(h_ref[...], w_ref[...], preferred_element_type=jnp.float32)
        + b_ref[...]
    )


def _sig(x):
    return 0.5 * jnp.tanh(0.5 * x) + 0.5


def _attn_body(xr_ref, xc_ref, d_ref, em_ref, wa_r_ref, wa_c_ref, wa_d_ref,
               ba1_ref, w2_ref, ba2_ref, w3_ref, ba3_ref, o_ref):
    bf = jnp.bfloat16
    xr = xr_ref[...]
    xc = xc_ref[...]
    em = em_ref[...]
    dm = d_ref[...] * em
    a1 = jnp.dot(xr.astype(bf), wa_r_ref[...].astype(bf),
                 preferred_element_type=jnp.float32)
    a1 = a1 + jnp.dot(xc.astype(bf), wa_c_ref[...].astype(bf),
                      preferred_element_type=jnp.float32)
    a1 = a1 + dm * wa_d_ref[...] + ba1_ref[...]
    a1 = a1 * _sig(a1)
    a2 = jnp.dot(a1.astype(bf), w2_ref[...].astype(bf),
                 preferred_element_type=jnp.float32)
    a2 = a2 + ba2_ref[...]
    a2 = a2 * _sig(a2)
    s = jnp.sum(a2 * w3_ref[...], axis=1, keepdims=True) + ba3_ref[...]
    att = _sig(s) * em
    o_ref[...] = xc * att


def _final_body(part_ref, x_ref, wn1_ref, bn1_ref, gn1_ref, bln1_ref,
                wn2_ref, bn2_ref, gln_ref, bln_ref, o_ref):
    acc = (part_ref[0] + part_ref[1]) * (1.0 / 100.0)
    t = jnp.dot(acc, wn1_ref[...], preferred_element_type=jnp.float32)
    t = t + bn1_ref[...]
    m = jnp.mean(t, axis=1, keepdims=True)
    v = jnp.mean((t - m) ** 2, axis=1, keepdims=True)
    t = (t - m) / jnp.sqrt(v + 1e-5) * gn1_ref[...] + bln1_ref[...]
    t = t * jax.nn.sigmoid(t)
    t = jnp.dot(t, wn2_ref[...], preferred_element_type=jnp.float32)
    o = t + bn2_ref[...] + x_ref[...]
    m = jnp.mean(o, axis=1, keepdims=True)
    v = jnp.mean((o - m) ** 2, axis=1, keepdims=True)
    o = (o - m) / jnp.sqrt(v + 1e-5) * gln_ref[...] + bln_ref[...]
    o_ref[...] = o * jax.nn.sigmoid(o)


def _make_gather(N, D, E):
    epw = E // _NW
    nch = epw // _C
    mesh = plsc.VectorSubcoreMesh(core_axis_name="c", subcore_axis_name="s")

    @functools.partial(
        pl.kernel,
        mesh=mesh,
        out_type=(
            jax.ShapeDtypeStruct((E, D), jnp.float32),
            jax.ShapeDtypeStruct((E, D), jnp.float32),
        ),
        scratch_types=[
            pltpu.VMEM((epw,), jnp.int32),
            pltpu.VMEM((epw,), jnp.int32),
            pltpu.VMEM((_C, D), jnp.float32),
            pltpu.VMEM((_C, D), jnp.float32),
            pltpu.SemaphoreType.DMA,
            pltpu.SemaphoreType.DMA,
        ],
    )
    def gather(x_hbm, row_hbm, col_hbm, xr_hbm, xc_hbm,
               idxr, idxc, bufr, bufc, semr, semc):
        wid = lax.axis_index("s") * _NC + lax.axis_index("c")
        base = wid * epw
        pltpu.sync_copy(row_hbm.at[pl.ds(base, epw)], idxr)
        pltpu.sync_copy(col_hbm.at[pl.ds(base, epw)], idxc)

        def chunk(j, carry):
            e0 = base + j * _C
            cr = pltpu.async_copy(x_hbm.at[idxr.at[pl.ds(j * _C, _C)]],
                                  bufr, semr)
            cc = pltpu.async_copy(x_hbm.at[idxc.at[pl.ds(j * _C, _C)]],
                                  bufc, semc)
            cr.wait()
            cc.wait()
            pltpu.sync_copy(bufr, xr_hbm.at[pl.ds(e0, _C)])
            pltpu.sync_copy(bufc, xc_hbm.at[pl.ds(e0, _C)])
            return carry

        lax.fori_loop(0, nch, chunk, 0)

    return gather


def _make_scatter(N, D, E, zrows):
    epw = E // _NW
    nch = epw // _C
    zlast = N - (_NS - 1) * zrows
    mesh = plsc.VectorSubcoreMesh(core_axis_name="c", subcore_axis_name="s")

    @functools.partial(
        pl.kernel,
        mesh=mesh,
        out_type=jax.ShapeDtypeStruct((_NC, N, D), jnp.float32),
        scratch_types=[
            pltpu.VMEM((_C,), jnp.int32),
            pltpu.VMEM((_C, D), jnp.float32),
            pltpu.VMEM_SHARED((N, D), jnp.float32),
        ],
    )
    def scatter(agg_hbm, row_hbm, zeros_hbm, part_hbm, idxb, buf, accum):
        c = lax.axis_index("c")
        s = lax.axis_index("s")
        wid = s * _NC + c

        @pl.when(s < _NS - 1)
        def _():
            pltpu.sync_copy(zeros_hbm.at[pl.ds(0, zrows)],
                            accum.at[pl.ds(s * zrows, zrows)])

        @pl.when(s == _NS - 1)
        def _():
            pltpu.sync_copy(zeros_hbm.at[pl.ds(0, zlast)],
                            accum.at[pl.ds((_NS - 1) * zrows, zlast)])

        plsc.subcore_barrier()
        base = wid * epw

        def chunk(j, carry):
            e0 = base + j * _C
            pltpu.sync_copy(row_hbm.at[pl.ds(e0, _C)], idxb)
            pltpu.sync_copy(agg_hbm.at[pl.ds(e0, _C)], buf)
            pltpu.sync_copy(buf, accum.at[idxb], add=True)
            return carry

        lax.fori_loop(0, nch, chunk, 0)
        plsc.subcore_barrier()

        @pl.when(s < _NS - 1)
        def _():
            pltpu.sync_copy(accum.at[pl.ds(s * zrows, zrows)],
                            part_hbm.at[c, pl.ds(s * zrows, zrows)])

        @pl.when(s == _NS - 1)
        def _():
            pltpu.sync_copy(accum.at[pl.ds((_NS - 1) * zrows, zlast)],
                            part_hbm.at[c, pl.ds((_NS - 1) * zrows, zlast)])

    return scatter


def kernel(h, distances, edges, node_mask, edge_mask, W1, b1, Wa1, ba1,
           Wa2, ba2, Wa3, ba3, Wn1, bn1, gn1, bln1, Wn2, bn2, g_ln, b_ln):
    N, D = h.shape
    E = edges.shape[1]
    D2 = 2 * D
    assert E % (_NW * _C) == 0 and N % 8 == 0

    row = edges[0]
    col = edges[1]

    tn = 1000
    gn = N // tn
    x = pl.pallas_call(
        _linear_body,
        grid=(gn,),
        in_specs=[
            pl.BlockSpec((tn, D), lambda i: (i, 0)),
            pl.BlockSpec((D, D), lambda i: (0, 0)),
            pl.BlockSpec((1, D), lambda i: (0, 0)),
        ],
        out_specs=pl.BlockSpec((tn, D), lambda i: (i, 0)),
        out_shape=jax.ShapeDtypeStruct((N, D), jnp.float32),
    )(h, W1, b1.reshape(1, D))

    xr, xc = _make_gather(N, D, E)(x, row, col)

    te = 1000
    ge = E // te
    wa_r = Wa1[:D]
    wa_c = Wa1[D:D2]
    wa_d = Wa1[D2:D2 + 1]
    agg = pl.pallas_call(
        _attn_body,
        grid=(ge,),
        in_specs=[
            pl.BlockSpec((te, D), lambda i: (i, 0)),
            pl.BlockSpec((te, D), lambda i: (i, 0)),
            pl.BlockSpec((te, 1), lambda i: (i, 0)),
            pl.BlockSpec((te, 1), lambda i: (i, 0)),
            pl.BlockSpec((D, D2), lambda i: (0, 0)),
            pl.BlockSpec((D, D2), lambda i: (0, 0)),
            pl.BlockSpec((1, D2), lambda i: (0, 0)),
            pl.BlockSpec((1, D2), lambda i: (0, 0)),
            pl.BlockSpec((D2, D), lambda i: (0, 0)),
            pl.BlockSpec((1, D), lambda i: (0, 0)),
            pl.BlockSpec((1, D), lambda i: (0, 0)),
            pl.BlockSpec((1, 1), lambda i: (0, 0)),
        ],
        out_specs=pl.BlockSpec((te, D), lambda i: (i, 0)),
        out_shape=jax.ShapeDtypeStruct((E, D), jnp.float32),
    )(xr, xc, distances, edge_mask, wa_r, wa_c, wa_d,
      ba1.reshape(1, D2), Wa2, ba2.reshape(1, D), Wa3.reshape(1, D),
      ba3.reshape(1, 1))

    zrows = 640
    zeros = jnp.zeros((zrows, D), jnp.float32)
    part = _make_scatter(N, D, E, zrows)(agg, row, zeros)

    out = pl.pallas_call(
        _final_body,
        grid=(gn,),
        in_specs=[
            pl.BlockSpec((_NC, tn, D), lambda i: (0, i, 0)),
            pl.BlockSpec((tn, D), lambda i: (i, 0)),
            pl.BlockSpec((D, D), lambda i: (0, 0)),
            pl.BlockSpec((1, D), lambda i: (0, 0)),
            pl.BlockSpec((1, D), lambda i: (0, 0)),
            pl.BlockSpec((1, D), lambda i: (0, 0)),
            pl.BlockSpec((D, D), lambda i: (0, 0)),
            pl.BlockSpec((1, D), lambda i: (0, 0)),
            pl.BlockSpec((1, D), lambda i: (0, 0)),
            pl.BlockSpec((1, D), lambda i: (0, 0)),
        ],
        out_specs=pl.BlockSpec((tn, D), lambda i: (i, 0)),
        out_shape=jax.ShapeDtypeStruct((N, D), jnp.float32),
    )(part, x, Wn1, bn1.reshape(1, D), gn1.reshape(1, D),
      bln1.reshape(1, D), Wn2, bn2.reshape(1, D), g_ln.reshape(1, D),
      b_ln.reshape(1, D))
    return out


# speedup vs baseline: 1.2416x; 1.2416x over previous
import functools

import jax
import jax.numpy as jnp
from jax import lax
from jax.experimental import pallas as pl
from jax.experimental.pallas import tpu as pltpu
from jax.experimental.pallas import tpu_sc as plsc

_NC = 2
_NS = 16
_NW = _NC * _NS
_C = 80


def _linear_body(h_ref, w_ref, b_ref, o_ref):
    o_ref[...] = (
        jnp.dot(h_ref[...], w_ref[...], preferred_element_type=jnp.float32)
        + b_ref[...]
    )


def _sig(x):
    return 0.5 * jnp.tanh(0.5 * x) + 0.5


def _attn_body(xr_ref, xc_ref, d_ref, em_ref, wa_r_ref, wa_c_ref, wa_d_ref,
               ba1_ref, w2_ref, ba2_ref, w3_ref, ba3_ref, o_ref):
    bf = jnp.bfloat16
    xr = xr_ref[...].astype(bf)
    xc = xc_ref[...]
    em = em_ref[...]
    dm = d_ref[...] * em
    a1 = jnp.dot(xr, wa_r_ref[...].astype(bf),
                 preferred_element_type=jnp.float32)
    a1 = a1 + jnp.dot(xc.astype(bf), wa_c_ref[...].astype(bf),
                      preferred_element_type=jnp.float32)
    a1 = a1 + dm * wa_d_ref[...] + ba1_ref[...]
    a1 = a1 * _sig(a1)
    a2 = jnp.dot(a1.astype(bf), w2_ref[...].astype(bf),
                 preferred_element_type=jnp.float32)
    a2 = a2 + ba2_ref[...]
    a2 = a2 * _sig(a2)
    s = jnp.sum(a2 * w3_ref[...], axis=1, keepdims=True) + ba3_ref[...]
    att = _sig(s) * em
    o_ref[...] = xc * att


def _final_body(part_ref, x_ref, wn1_ref, bn1_ref, gn1_ref, bln1_ref,
                wn2_ref, bn2_ref, gln_ref, bln_ref, o_ref):
    acc = (part_ref[0] + part_ref[1]) * (1.0 / 100.0)
    t = jnp.dot(acc, wn1_ref[...], preferred_element_type=jnp.float32)
    t = t + bn1_ref[...]
    m = jnp.mean(t, axis=1, keepdims=True)
    v = jnp.mean((t - m) ** 2, axis=1, keepdims=True)
    t = (t - m) / jnp.sqrt(v + 1e-5) * gn1_ref[...] + bln1_ref[...]
    t = t * jax.nn.sigmoid(t)
    t = jnp.dot(t, wn2_ref[...], preferred_element_type=jnp.float32)
    o = t + bn2_ref[...] + x_ref[...]
    m = jnp.mean(o, axis=1, keepdims=True)
    v = jnp.mean((o - m) ** 2, axis=1, keepdims=True)
    o = (o - m) / jnp.sqrt(v + 1e-5) * gln_ref[...] + bln_ref[...]
    o_ref[...] = o * jax.nn.sigmoid(o)


def _make_gather(N, D, e0, elen):
    epw = elen // _NW
    nch = epw // _C
    mesh = plsc.VectorSubcoreMesh(core_axis_name="c", subcore_axis_name="s")

    @functools.partial(
        pl.kernel,
        mesh=mesh,
        out_type=(
            jax.ShapeDtypeStruct((elen, D), jnp.float32),
            jax.ShapeDtypeStruct((elen, D), jnp.float32),
        ),
        scratch_types=[
            pltpu.VMEM((epw,), jnp.int32),
            pltpu.VMEM((epw,), jnp.int32),
            pltpu.VMEM((2, _C, D), jnp.float32),
            pltpu.VMEM((2, _C, D), jnp.float32),
            pltpu.SemaphoreType.DMA,
            pltpu.SemaphoreType.DMA,
            pltpu.SemaphoreType.DMA,
            pltpu.SemaphoreType.DMA,
        ],
    )
    def gather(x_hbm, row_hbm, col_hbm, xr_hbm, xc_hbm,
               idxr, idxc, bufr, bufc, g0, g1, w0, w1):
        gsem = (g0, g1)
        wsem = (w0, w1)
        wid = lax.axis_index("s") * _NC + lax.axis_index("c")
        base = wid * epw
        pltpu.sync_copy(row_hbm.at[pl.ds(e0 + base, epw)], idxr)
        pltpu.sync_copy(col_hbm.at[pl.ds(e0 + base, epw)], idxc)

        def start_gather(j, b):
            pltpu.async_copy(x_hbm.at[idxr.at[pl.ds(j * _C, _C)]],
                             bufr.at[b], gsem[b])
            pltpu.async_copy(x_hbm.at[idxc.at[pl.ds(j * _C, _C)]],
                             bufc.at[b], gsem[b])

        def wait_gather(j, b):
            pltpu.make_async_copy(x_hbm.at[idxr.at[pl.ds(j * _C, _C)]],
                                  bufr.at[b], gsem[b]).wait()
            pltpu.make_async_copy(x_hbm.at[idxc.at[pl.ds(j * _C, _C)]],
                                  bufc.at[b], gsem[b]).wait()

        def start_write(j, b):
            o0 = base + j * _C
            pltpu.async_copy(bufr.at[b], xr_hbm.at[pl.ds(o0, _C)], wsem[b])
            pltpu.async_copy(bufc.at[b], xc_hbm.at[pl.ds(o0, _C)], wsem[b])

        def wait_write(j, b):
            o0 = base + j * _C
            pltpu.make_async_copy(bufr.at[b], xr_hbm.at[pl.ds(o0, _C)],
                                  wsem[b]).wait()
            pltpu.make_async_copy(bufc.at[b], xc_hbm.at[pl.ds(o0, _C)],
                                  wsem[b]).wait()

        start_gather(0, 0)

        def outer(g, carry):
            for b in (0, 1):
                j = g * 2 + b

                @pl.when(j >= 1)
                def _():
                    wait_write(j - 1, 1 - b)

                @pl.when(j + 1 < nch)
                def _():
                    start_gather(j + 1, 1 - b)

                wait_gather(j, b)
                start_write(j, b)
            return carry

        lax.fori_loop(0, nch // 2, outer, 0)
        if nch % 2 == 1:
            wait_write(nch - 2, 1)
            wait_gather(nch - 1, 0)
            start_write(nch - 1, 0)
            wait_write(nch - 1, 0)
        else:
            wait_write(nch - 1, 1)

    return gather


def _make_scatter(N, D, e0, elen, zrows, chain):
    epw = elen // _NW
    nch = epw // _C
    zlast = N - (_NS - 1) * zrows
    mesh = plsc.VectorSubcoreMesh(core_axis_name="c", subcore_axis_name="s")

    @functools.partial(
        pl.kernel,
        mesh=mesh,
        out_type=jax.ShapeDtypeStruct((_NC, N, D), jnp.float32),
        scratch_types=[
            pltpu.VMEM((2, _C), jnp.int32),
            pltpu.VMEM((2, _C, D), jnp.float32),
            pltpu.VMEM_SHARED((N, D), jnp.float32),
            pltpu.SemaphoreType.DMA,
            pltpu.SemaphoreType.DMA,
        ],
    )
    def scatter(agg_hbm, row_hbm, init_hbm, part_hbm, idxb, buf, accum,
                l0, l1):
        c = lax.axis_index("c")
        s = lax.axis_index("s")
        wid = s * _NC + c

        @pl.when(s < _NS - 1)
        def _():
            if chain:
                pltpu.sync_copy(init_hbm.at[c, pl.ds(s * zrows, zrows)],
                                accum.at[pl.ds(s * zrows, zrows)])
            else:
                pltpu.sync_copy(init_hbm.at[pl.ds(0, zrows)],
                                accum.at[pl.ds(s * zrows, zrows)])

        @pl.when(s == _NS - 1)
        def _():
            if chain:
                pltpu.sync_copy(
                    init_hbm.at[c, pl.ds((_NS - 1) * zrows, zlast)],
                    accum.at[pl.ds((_NS - 1) * zrows, zlast)])
            else:
                pltpu.sync_copy(init_hbm.at[pl.ds(0, zlast)],
                                accum.at[pl.ds((_NS - 1) * zrows, zlast)])

        plsc.subcore_barrier()
        base = wid * epw
        lsem = (l0, l1)

        def start_load(j, b):
            o0 = base + j * _C
            pltpu.async_copy(row_hbm.at[pl.ds(e0 + o0, _C)], idxb.at[b],
                             lsem[b])
            pltpu.async_copy(agg_hbm.at[pl.ds(o0, _C)], buf.at[b], lsem[b])

        def wait_load(j, b):
            o0 = base + j * _C
            pltpu.make_async_copy(row_hbm.at[pl.ds(e0 + o0, _C)], idxb.at[b],
                                  lsem[b]).wait()
            pltpu.make_async_copy(agg_hbm.at[pl.ds(o0, _C)], buf.at[b],
                                  lsem[b]).wait()

        start_load(0, 0)

        def outer(g, carry):
            for b in (0, 1):
                j = g * 2 + b

                @pl.when(j + 1 < nch)
                def _():
                    start_load(j + 1, 1 - b)

                wait_load(j, b)
                pltpu.sync_copy(buf.at[b], accum.at[idxb.at[b]], add=True)
            return carry

        lax.fori_loop(0, nch // 2, outer, 0)
        if nch % 2 == 1:
            wait_load(nch - 1, 0)
            pltpu.sync_copy(buf.at[0], accum.at[idxb.at[0]], add=True)
        plsc.subcore_barrier()

        @pl.when(s < _NS - 1)
        def _():
            pltpu.sync_copy(accum.at[pl.ds(s * zrows, zrows)],
                            part_hbm.at[c, pl.ds(s * zrows, zrows)])

        @pl.when(s == _NS - 1)
        def _():
            pltpu.sync_copy(accum.at[pl.ds((_NS - 1) * zrows, zlast)],
                            part_hbm.at[c, pl.ds((_NS - 1) * zrows, zlast)])

    return scatter


def kernel(h, distances, edges, node_mask, edge_mask, W1, b1, Wa1, ba1,
           Wa2, ba2, Wa3, ba3, Wn1, bn1, gn1, bln1, Wn2, bn2, g_ln, b_ln):
    N, D = h.shape
    E = edges.shape[1]
    D2 = 2 * D
    assert E % (5 * _NW * _C) == 0 and N % 8 == 0

    row = edges[0]
    col = edges[1]

    tn = 1000
    gn = N // tn
    x = pl.pallas_call(
        _linear_body,
        grid=(gn,),
        in_specs=[
            pl.BlockSpec((tn, D), lambda i: (i, 0)),
            pl.BlockSpec((D, D), lambda i: (0, 0)),
            pl.BlockSpec((1, D), lambda i: (0, 0)),
        ],
        out_specs=pl.BlockSpec((tn, D), lambda i: (i, 0)),
        out_shape=jax.ShapeDtypeStruct((N, D), jnp.float32),
    )(h, W1, b1.reshape(1, D))

    K = 5
    es = E // K
    te = 1000
    ge = es // te
    wa_r = Wa1[:D]
    wa_c = Wa1[D:D2]
    wa_d = Wa1[D2:D2 + 1]
    zrows = 640
    zeros = jnp.zeros((zrows, D), jnp.float32)
    part = zeros

    for k in range(K):
        xr, xc = _make_gather(N, D, k * es, es)(x, row, col)
        off = k * ge
        agg = pl.pallas_call(
            _attn_body,
            grid=(ge,),
            in_specs=[
                pl.BlockSpec((te, D), lambda i: (i, 0)),
                pl.BlockSpec((te, D), lambda i: (i, 0)),
                pl.BlockSpec((te, 1), lambda i, o=off: (i + o, 0)),
                pl.BlockSpec((te, 1), lambda i, o=off: (i + o, 0)),
                pl.BlockSpec((D, D2), lambda i: (0, 0)),
                pl.BlockSpec((D, D2), lambda i: (0, 0)),
                pl.BlockSpec((1, D2), lambda i: (0, 0)),
                pl.BlockSpec((1, D2), lambda i: (0, 0)),
                pl.BlockSpec((D2, D), lambda i: (0, 0)),
                pl.BlockSpec((1, D), lambda i: (0, 0)),
                pl.BlockSpec((1, D), lambda i: (0, 0)),
                pl.BlockSpec((1, 1), lambda i: (0, 0)),
            ],
            out_specs=pl.BlockSpec((te, D), lambda i: (i, 0)),
            out_shape=jax.ShapeDtypeStruct((es, D), jnp.float32),
        )(xr, xc, distances, edge_mask, wa_r, wa_c, wa_d,
          ba1.reshape(1, D2), Wa2, ba2.reshape(1, D), Wa3.reshape(1, D),
          ba3.reshape(1, 1))
        part = _make_scatter(N, D, k * es, es, zrows, chain=(k > 0))(
            agg, row, part)

    out = pl.pallas_call(
        _final_body,
        grid=(gn,),
        in_specs=[
            pl.BlockSpec((_NC, tn, D), lambda i: (0, i, 0)),
            pl.BlockSpec((tn, D), lambda i: (i, 0)),
            pl.BlockSpec((D, D), lambda i: (0, 0)),
            pl.BlockSpec((1, D), lambda i: (0, 0)),
            pl.BlockSpec((1, D), lambda i: (0, 0)),
            pl.BlockSpec((1, D), lambda i: (0, 0)),
            pl.BlockSpec((D, D), lambda i: (0, 0)),
            pl.BlockSpec((1, D), lambda i: (0, 0)),
            pl.BlockSpec((1, D), lambda i: (0, 0)),
            pl.BlockSpec((1, D), lambda i: (0, 0)),
        ],
        out_specs=pl.BlockSpec((tn, D), lambda i: (i, 0)),
        out_shape=jax.ShapeDtypeStruct((N, D), jnp.float32),
    )(part, x, Wn1, bn1.reshape(1, D), gn1.reshape(1, D),
      bln1.reshape(1, D), Wn2, bn2.reshape(1, D), g_ln.reshape(1, D),
      b_ln.reshape(1, D))
    return out


# speedup vs baseline: 1.4007x; 1.1281x over previous
import functools

import jax
import jax.numpy as jnp
from jax import lax
from jax.experimental import pallas as pl
from jax.experimental.pallas import tpu as pltpu
from jax.experimental.pallas import tpu_sc as plsc

_NC = 2
_NS = 16
_NW = _NC * _NS
_C = 80


def _linear_body(h_ref, w_ref, b_ref, o_ref):
    o_ref[...] = (
        jnp.dot(h_ref[...], w_ref[...], preferred_element_type=jnp.float32)
        + b_ref[...]
    )


def _sig(x):
    return 0.5 * jnp.tanh(0.5 * x) + 0.5


def _attn_body(xr_ref, xc_ref, d_ref, em_ref, wa_r_ref, wa_c_ref, wa_d_ref,
               ba1_ref, w2_ref, ba2_ref, w3_ref, ba3_ref, o_ref):
    bf = jnp.bfloat16
    xr = xr_ref[...].astype(bf)
    xc = xc_ref[...]
    em = jnp.reshape(em_ref[...], (em_ref.shape[0], 1))
    dm = jnp.reshape(d_ref[...], (d_ref.shape[0], 1))
    a1 = jnp.dot(xr, wa_r_ref[...].astype(bf),
                 preferred_element_type=jnp.float32)
    a1 = a1 + jnp.dot(xc.astype(bf), wa_c_ref[...].astype(bf),
                      preferred_element_type=jnp.float32)
    a1 = a1 + dm * wa_d_ref[...] + ba1_ref[...]
    a1 = a1 * _sig(a1)
    a2 = jnp.dot(a1.astype(bf), w2_ref[...].astype(bf),
                 preferred_element_type=jnp.float32)
    a2 = a2 + ba2_ref[...]
    a2 = a2 * _sig(a2)
    s = jnp.sum(a2 * w3_ref[...], axis=1, keepdims=True) + ba3_ref[...]
    att = _sig(s) * em
    o_ref[...] = xc * att


def _final_body(part_ref, x_ref, wn1_ref, bn1_ref, gn1_ref, bln1_ref,
                wn2_ref, bn2_ref, gln_ref, bln_ref, o_ref):
    acc = (part_ref[0] + part_ref[1]) * (1.0 / 100.0)
    t = jnp.dot(acc, wn1_ref[...], preferred_element_type=jnp.float32)
    t = t + bn1_ref[...]
    m = jnp.mean(t, axis=1, keepdims=True)
    v = jnp.mean((t - m) ** 2, axis=1, keepdims=True)
    t = (t - m) / jnp.sqrt(v + 1e-5) * gn1_ref[...] + bln1_ref[...]
    t = t * jax.nn.sigmoid(t)
    t = jnp.dot(t, wn2_ref[...], preferred_element_type=jnp.float32)
    o = t + bn2_ref[...] + x_ref[...]
    m = jnp.mean(o, axis=1, keepdims=True)
    v = jnp.mean((o - m) ** 2, axis=1, keepdims=True)
    o = (o - m) / jnp.sqrt(v + 1e-5) * gln_ref[...] + bln_ref[...]
    o_ref[...] = o * jax.nn.sigmoid(o)


def _make_gather(N, D, e0, elen):
    epw = elen // _NW
    nch = epw // _C
    mesh = plsc.VectorSubcoreMesh(core_axis_name="c", subcore_axis_name="s")

    @functools.partial(
        pl.kernel,
        mesh=mesh,
        out_type=(
            jax.ShapeDtypeStruct((elen, D), jnp.float32),
            jax.ShapeDtypeStruct((elen, D), jnp.float32),
        ),
        scratch_types=[
            pltpu.VMEM((epw,), jnp.int32),
            pltpu.VMEM((epw,), jnp.int32),
            pltpu.VMEM((2, _C, D), jnp.float32),
            pltpu.VMEM((2, _C, D), jnp.float32),
            pltpu.SemaphoreType.DMA,
            pltpu.SemaphoreType.DMA,
            pltpu.SemaphoreType.DMA,
            pltpu.SemaphoreType.DMA,
        ],
    )
    def gather(x_hbm, row_hbm, col_hbm, xr_hbm, xc_hbm,
               idxr, idxc, bufr, bufc, g0, g1, w0, w1):
        gsem = (g0, g1)
        wsem = (w0, w1)
        wid = lax.axis_index("s") * _NC + lax.axis_index("c")
        base = wid * epw
        pltpu.sync_copy(row_hbm.at[pl.ds(e0 + base, epw)], idxr)
        pltpu.sync_copy(col_hbm.at[pl.ds(e0 + base, epw)], idxc)

        def start_gather(j, b):
            pltpu.async_copy(x_hbm.at[idxr.at[pl.ds(j * _C, _C)]],
                             bufr.at[b], gsem[b])
            pltpu.async_copy(x_hbm.at[idxc.at[pl.ds(j * _C, _C)]],
                             bufc.at[b], gsem[b])

        def wait_gather(j, b):
            pltpu.make_async_copy(x_hbm.at[idxr.at[pl.ds(j * _C, _C)]],
                                  bufr.at[b], gsem[b]).wait()
            pltpu.make_async_copy(x_hbm.at[idxc.at[pl.ds(j * _C, _C)]],
                                  bufc.at[b], gsem[b]).wait()

        def start_write(j, b):
            o0 = base + j * _C
            pltpu.async_copy(bufr.at[b], xr_hbm.at[pl.ds(o0, _C)], wsem[b])
            pltpu.async_copy(bufc.at[b], xc_hbm.at[pl.ds(o0, _C)], wsem[b])

        def wait_write(j, b):
            o0 = base + j * _C
            pltpu.make_async_copy(bufr.at[b], xr_hbm.at[pl.ds(o0, _C)],
                                  wsem[b]).wait()
            pltpu.make_async_copy(bufc.at[b], xc_hbm.at[pl.ds(o0, _C)],
                                  wsem[b]).wait()

        start_gather(0, 0)

        def outer(g, carry):
            for b in (0, 1):
                j = g * 2 + b

                @pl.when(j >= 1)
                def _():
                    wait_write(j - 1, 1 - b)

                @pl.when(j + 1 < nch)
                def _():
                    start_gather(j + 1, 1 - b)

                wait_gather(j, b)
                start_write(j, b)
            return carry

        lax.fori_loop(0, nch // 2, outer, 0)
        if nch % 2 == 1:
            wait_write(nch - 2, 1)
            wait_gather(nch - 1, 0)
            start_write(nch - 1, 0)
            wait_write(nch - 1, 0)
        else:
            wait_write(nch - 1, 1)

    return gather


def _make_scatter(N, D, e0, elen, zrows, chain):
    epw = elen // _NW
    nch = epw // _C
    zlast = N - (_NS - 1) * zrows
    mesh = plsc.VectorSubcoreMesh(core_axis_name="c", subcore_axis_name="s")

    @functools.partial(
        pl.kernel,
        mesh=mesh,
        out_type=jax.ShapeDtypeStruct((_NC, N, D), jnp.float32),
        scratch_types=[
            pltpu.VMEM((2, _C), jnp.int32),
            pltpu.VMEM((2, _C, D), jnp.float32),
            pltpu.VMEM_SHARED((N, D), jnp.float32),
            pltpu.SemaphoreType.DMA,
            pltpu.SemaphoreType.DMA,
        ],
    )
    def scatter(agg_hbm, row_hbm, init_hbm, part_hbm, idxb, buf, accum,
                l0, l1):
        c = lax.axis_index("c")
        s = lax.axis_index("s")
        wid = s * _NC + c

        @pl.when(s < _NS - 1)
        def _():
            if chain:
                pltpu.sync_copy(init_hbm.at[c, pl.ds(s * zrows, zrows)],
                                accum.at[pl.ds(s * zrows, zrows)])
            else:
                pltpu.sync_copy(init_hbm.at[pl.ds(0, zrows)],
                                accum.at[pl.ds(s * zrows, zrows)])

        @pl.when(s == _NS - 1)
        def _():
            if chain:
                pltpu.sync_copy(
                    init_hbm.at[c, pl.ds((_NS - 1) * zrows, zlast)],
                    accum.at[pl.ds((_NS - 1) * zrows, zlast)])
            else:
                pltpu.sync_copy(init_hbm.at[pl.ds(0, zlast)],
                                accum.at[pl.ds((_NS - 1) * zrows, zlast)])

        plsc.subcore_barrier()
        base = wid * epw
        lsem = (l0, l1)

        def start_load(j, b):
            o0 = base + j * _C
            pltpu.async_copy(row_hbm.at[pl.ds(e0 + o0, _C)], idxb.at[b],
                             lsem[b])
            pltpu.async_copy(agg_hbm.at[pl.ds(o0, _C)], buf.at[b], lsem[b])

        def wait_load(j, b):
            o0 = base + j * _C
            pltpu.make_async_copy(row_hbm.at[pl.ds(e0 + o0, _C)], idxb.at[b],
                                  lsem[b]).wait()
            pltpu.make_async_copy(agg_hbm.at[pl.ds(o0, _C)], buf.at[b],
                                  lsem[b]).wait()

        start_load(0, 0)

        def outer(g, carry):
            for b in (0, 1):
                j = g * 2 + b

                @pl.when(j + 1 < nch)
                def _():
                    start_load(j + 1, 1 - b)

                wait_load(j, b)
                pltpu.sync_copy(buf.at[b], accum.at[idxb.at[b]], add=True)
            return carry

        lax.fori_loop(0, nch // 2, outer, 0)
        if nch % 2 == 1:
            wait_load(nch - 1, 0)
            pltpu.sync_copy(buf.at[0], accum.at[idxb.at[0]], add=True)
        plsc.subcore_barrier()

        @pl.when(s < _NS - 1)
        def _():
            pltpu.sync_copy(accum.at[pl.ds(s * zrows, zrows)],
                            part_hbm.at[c, pl.ds(s * zrows, zrows)])

        @pl.when(s == _NS - 1)
        def _():
            pltpu.sync_copy(accum.at[pl.ds((_NS - 1) * zrows, zlast)],
                            part_hbm.at[c, pl.ds((_NS - 1) * zrows, zlast)])

    return scatter


def kernel(h, distances, edges, node_mask, edge_mask, W1, b1, Wa1, ba1,
           Wa2, ba2, Wa3, ba3, Wn1, bn1, gn1, bln1, Wn2, bn2, g_ln, b_ln):
    N, D = h.shape
    E = edges.shape[1]
    D2 = 2 * D
    assert E % (5 * _NW * _C) == 0 and N % 8 == 0

    row = edges[0]
    col = edges[1]

    tn = 1000
    gn = N // tn
    x = pl.pallas_call(
        _linear_body,
        grid=(gn,),
        in_specs=[
            pl.BlockSpec((tn, D), lambda i: (i, 0)),
            pl.BlockSpec((D, D), lambda i: (0, 0)),
            pl.BlockSpec((1, D), lambda i: (0, 0)),
        ],
        out_specs=pl.BlockSpec((tn, D), lambda i: (i, 0)),
        out_shape=jax.ShapeDtypeStruct((N, D), jnp.float32),
    )(h, W1, b1.reshape(1, D))

    K = 5
    es = E // K
    te = 512
    ge = es // te
    dm1 = (distances * edge_mask).reshape(E)
    em1 = edge_mask.reshape(E)
    wa_r = Wa1[:D]
    wa_c = Wa1[D:D2]
    wa_d = Wa1[D2:D2 + 1]
    zrows = 640
    zeros = jnp.zeros((zrows, D), jnp.float32)
    part = zeros

    for k in range(K):
        xr, xc = _make_gather(N, D, k * es, es)(x, row, col)
        off = k * ge
        agg = pl.pallas_call(
            _attn_body,
            grid=(ge,),
            in_specs=[
                pl.BlockSpec((te, D), lambda i: (i, 0)),
                pl.BlockSpec((te, D), lambda i: (i, 0)),
                pl.BlockSpec((te,), lambda i, o=off: (i + o,)),
                pl.BlockSpec((te,), lambda i, o=off: (i + o,)),
                pl.BlockSpec((D, D2), lambda i: (0, 0)),
                pl.BlockSpec((D, D2), lambda i: (0, 0)),
                pl.BlockSpec((1, D2), lambda i: (0, 0)),
                pl.BlockSpec((1, D2), lambda i: (0, 0)),
                pl.BlockSpec((D2, D), lambda i: (0, 0)),
                pl.BlockSpec((1, D), lambda i: (0, 0)),
                pl.BlockSpec((1, D), lambda i: (0, 0)),
                pl.BlockSpec((1, 1), lambda i: (0, 0)),
            ],
            out_specs=pl.BlockSpec((te, D), lambda i: (i, 0)),
            out_shape=jax.ShapeDtypeStruct((es, D), jnp.float32),
        )(xr, xc, dm1, em1, wa_r, wa_c, wa_d,
          ba1.reshape(1, D2), Wa2, ba2.reshape(1, D), Wa3.reshape(1, D),
          ba3.reshape(1, 1))
        part = _make_scatter(N, D, k * es, es, zrows, chain=(k > 0))(
            agg, row, part)

    out = pl.pallas_call(
        _final_body,
        grid=(gn,),
        in_specs=[
            pl.BlockSpec((_NC, tn, D), lambda i: (0, i, 0)),
            pl.BlockSpec((tn, D), lambda i: (i, 0)),
            pl.BlockSpec((D, D), lambda i: (0, 0)),
            pl.BlockSpec((1, D), lambda i: (0, 0)),
            pl.BlockSpec((1, D), lambda i: (0, 0)),
            pl.BlockSpec((1, D), lambda i: (0, 0)),
            pl.BlockSpec((D, D), lambda i: (0, 0)),
            pl.BlockSpec((1, D), lambda i: (0, 0)),
            pl.BlockSpec((1, D), lambda i: (0, 0)),
            pl.BlockSpec((1, D), lambda i: (0, 0)),
        ],
        out_specs=pl.BlockSpec((tn, D), lambda i: (i, 0)),
        out_shape=jax.ShapeDtypeStruct((N, D), jnp.float32),
    )(part, x, Wn1, bn1.reshape(1, D), gn1.reshape(1, D),
      bln1.reshape(1, D), Wn2, bn2.reshape(1, D), g_ln.reshape(1, D),
      b_ln.reshape(1, D))
    return out
